# baseline (speedup 1.0000x reference)
import jax
import jax.numpy as jnp
from jax import lax
from jax.experimental import pallas as pl
from jax.experimental.pallas import tpu as pltpu
from jax.experimental.pallas import tpu_sc as plsc

N_NODES = 10000
D = 128
DP = D // 2
N_EDGES = 320000
C = 128
N_CHUNKS = N_EDGES // C
N_WORKERS = 32
BASE_CHUNKS = N_CHUNKS // N_WORKERS
EXTRA = N_CHUNKS - BASE_CHUNKS * N_WORKERS
GROUPS = C // 16
PAIRS = BASE_CHUNKS // 2


def _sc_kernel(table_hbm, edges_hbm, out_hbm,
               table_sh, idx0, idx1, s0, s1, d0, d1, o0, o1,
               sem_g, sem_idx, sem_out0, sem_out1, sem_st):
    nc = 2
    wid = lax.axis_index("s") * nc + lax.axis_index("c")
    my_chunks = BASE_CHUNKS + jnp.where(wid < EXTRA, 1, 0)
    start = BASE_CHUNKS * wid + jnp.minimum(wid, EXTRA)

    sid = lax.axis_index("s")
    rows_per_sub = N_NODES // 16
    pltpu.async_copy(table_hbm.at[pl.ds(sid * rows_per_sub, rows_per_sub)],
                     table_sh.at[pl.ds(sid * rows_per_sub, rows_per_sub)],
                     sem_st).wait()
    plsc.subcore_barrier()

    lane = lax.iota(jnp.int32, 16)
    idx_b = (idx0, idx1)
    s_b = (s0, s1)
    d_b = (d0, d1)
    o_b = (o0, o1)

    def issue_idx(c, b):
        pltpu.async_copy(edges_hbm.at[:, pl.ds((start + c) * C, C)],
                         idx_b[b], sem_idx)

    def wait_idx(b):
        pltpu.make_async_copy(edges_hbm.at[:, pl.ds(0, C)], idx_b[b],
                              sem_idx).wait()

    def launch_gathers(b):
        pltpu.async_copy(table_sh.at[idx_b[b].at[0]], s_b[b], sem_g)
        pltpu.async_copy(table_sh.at[idx_b[b].at[1]], d_b[b], sem_g)

    def wait_gathers(b):
        pltpu.make_async_copy(table_hbm.at[pl.ds(0, C)], s_b[b], sem_g).wait()
        pltpu.make_async_copy(table_hbm.at[pl.ds(0, C)], d_b[b], sem_g).wait()

    sem_out = (sem_out0, sem_out1)

    def drain_out(b):
        pltpu.make_async_copy(o_b[b], out_hbm.at[pl.ds(0, C)],
                              sem_out[b]).wait()

    def compute(c, b):
        s_rows, d_rows, out_v = s_b[b], d_b[b], o_b[b]

        @pl.when(c >= 2)
        def _():
            drain_out(b)

        for g in range(GROUPS):
            row_idx = g * 16 + lane

            def jstep(j, acc):
                col = (j & ~15) + ((j + lane) & 15)
                sv = plsc.load_gather(s_rows, [row_idx, col])
                dv = plsc.load_gather(d_rows, [row_idx, col])
                sa, sb = plsc.unpack(plsc.bitcast(sv, jnp.bfloat16),
                                     format=plsc.PackFormat.INTERLEAVED)
                da, db = plsc.unpack(plsc.bitcast(dv, jnp.bfloat16),
                                     format=plsc.PackFormat.INTERLEAVED)
                return acc + sa * da + sb * db

            acc = lax.fori_loop(0, DP, jstep, jnp.zeros((16,), jnp.float32),
                                unroll=8)
            p = 1.0 / (1.0 + jnp.exp(-acc))
            out_v[pl.ds(g * 16, 16)] = p

        pltpu.async_copy(out_v, out_hbm.at[pl.ds((start + c) * C, C)],
                         sem_out[b])

    def step(c, b, nb):
        wait_gathers(b)

        @pl.when(c + 1 < my_chunks)
        def _():
            wait_idx(nb)
            launch_gathers(nb)

        @pl.when(c + 2 < my_chunks)
        def _():
            issue_idx(c + 2, b)

        compute(c, b)

    pltpu.sync_copy(edges_hbm.at[:, pl.ds(start * C, C)], idx0)
    launch_gathers(0)
    issue_idx(1, 1)

    def pair_body(p, carry):
        step(2 * p, 0, 1)
        step(2 * p + 1, 1, 0)
        return carry

    lax.fori_loop(0, PAIRS, pair_body, 0)

    @pl.when(my_chunks > BASE_CHUNKS)
    def _():
        step(BASE_CHUNKS, 0, 1)

    drain_out(0)
    drain_out(1)


@jax.jit
def _run(table_pk, edge_index):
    mesh = plsc.VectorSubcoreMesh(core_axis_name="c", subcore_axis_name="s")
    kfn = pl.kernel(
        _sc_kernel,
        mesh=mesh,
        compiler_params=pltpu.CompilerParams(
            use_tc_tiling_on_sc=False, needs_layout_passes=False),
        out_type=jax.ShapeDtypeStruct((N_EDGES,), jnp.float32),
        scratch_types=[
            pltpu.VMEM_SHARED((N_NODES, DP), jnp.int32),
            pltpu.VMEM((2, C), jnp.int32),
            pltpu.VMEM((2, C), jnp.int32),
            pltpu.VMEM((C, DP), jnp.int32),
            pltpu.VMEM((C, DP), jnp.int32),
            pltpu.VMEM((C, DP), jnp.int32),
            pltpu.VMEM((C, DP), jnp.int32),
            pltpu.VMEM((C,), jnp.float32),
            pltpu.VMEM((C,), jnp.float32),
            pltpu.SemaphoreType.DMA,
            pltpu.SemaphoreType.DMA,
            pltpu.SemaphoreType.DMA,
            pltpu.SemaphoreType.DMA,
            pltpu.SemaphoreType.DMA,
        ],
    )
    return kfn(table_pk, edge_index)


def kernel(node_embedding_matrix, batch_x_index, edge_index):
    del batch_x_index
    table_bf = node_embedding_matrix.astype(jnp.bfloat16)
    table_pk = jax.lax.bitcast_convert_type(
        table_bf.reshape(N_NODES, DP, 2), jnp.int32)
    return _run(table_pk, edge_index)


# speedup vs baseline: 1.4245x; 1.4245x over previous
import jax
import jax.numpy as jnp
from jax import lax
from jax.experimental import pallas as pl
from jax.experimental.pallas import tpu as pltpu
from jax.experimental.pallas import tpu_sc as plsc

N_NODES = 10000
D = 128
DP = D // 2
N_EDGES = 320000
C = 128
N_CHUNKS = N_EDGES // C
N_WORKERS = 32
BASE_CHUNKS = N_CHUNKS // N_WORKERS
EXTRA = N_CHUNKS - BASE_CHUNKS * N_WORKERS
GROUPS = C // 16
PAIRS = BASE_CHUNKS // 2


def _sc_kernel(table_hbm, edges_hbm, out_hbm,
               table_sh, idx0, idx1, s0, s1, d0, d1, o0, o1,
               sem_g, sem_idx, sem_out0, sem_out1, sem_st):
    nc = 2
    wid = lax.axis_index("s") * nc + lax.axis_index("c")
    my_chunks = BASE_CHUNKS + jnp.where(wid < EXTRA, 1, 0)
    start = BASE_CHUNKS * wid + jnp.minimum(wid, EXTRA)

    sid = lax.axis_index("s")
    rows_per_sub = N_NODES // 16
    pltpu.async_copy(table_hbm.at[pl.ds(sid * rows_per_sub, rows_per_sub)],
                     table_sh.at[pl.ds(sid * rows_per_sub, rows_per_sub)],
                     sem_st).wait()
    plsc.subcore_barrier()

    lane = lax.iota(jnp.int32, 16)
    idx_b = (idx0, idx1)
    s_b = (s0, s1)
    d_b = (d0, d1)
    o_b = (o0, o1)

    def issue_idx(c, b):
        pltpu.async_copy(edges_hbm.at[:, pl.ds((start + c) * C, C)],
                         idx_b[b], sem_idx)

    def wait_idx(b):
        pltpu.make_async_copy(edges_hbm.at[:, pl.ds(0, C)], idx_b[b],
                              sem_idx).wait()

    def launch_gathers(b):
        pltpu.async_copy(table_sh.at[idx_b[b].at[0]], s_b[b], sem_g)
        pltpu.async_copy(table_sh.at[idx_b[b].at[1]], d_b[b], sem_g)

    def wait_gathers(b):
        pltpu.make_async_copy(table_hbm.at[pl.ds(0, C)], s_b[b], sem_g).wait()
        pltpu.make_async_copy(table_hbm.at[pl.ds(0, C)], d_b[b], sem_g).wait()

    sem_out = (sem_out0, sem_out1)

    def drain_out(b):
        pltpu.make_async_copy(o_b[b], out_hbm.at[pl.ds(0, C)],
                              sem_out[b]).wait()

    def compute(c, b):
        s_rows, d_rows, out_v = s_b[b], d_b[b], o_b[b]

        @pl.when(c >= 2)
        def _():
            drain_out(b)

        for g in range(GROUPS):
            row_idx = g * 16 + lane

            def term(j):
                col = (j & ~15) + ((j + lane) & 15)
                sv = plsc.load_gather(s_rows, [row_idx, col])
                dv = plsc.load_gather(d_rows, [row_idx, col])
                p = plsc.bitcast(sv, jnp.bfloat16) * plsc.bitcast(
                    dv, jnp.bfloat16)
                return plsc.unpack(p, format=plsc.PackFormat.INTERLEAVED)

            def jstep(t, carry):
                a0, a1, a2, a3 = carry
                pa0, pb0 = term(2 * t)
                pa1, pb1 = term(2 * t + 1)
                return a0 + pa0, a1 + pb0, a2 + pa1, a3 + pb1

            zero = jnp.zeros((16,), jnp.float32)
            a0, a1, a2, a3 = lax.fori_loop(0, DP // 2, jstep,
                                           (zero, zero, zero, zero),
                                           unroll=4)
            acc = (a0 + a1) + (a2 + a3)
            p = 1.0 / (1.0 + jnp.exp(-acc))
            out_v[pl.ds(g * 16, 16)] = p

        pltpu.async_copy(out_v, out_hbm.at[pl.ds((start + c) * C, C)],
                         sem_out[b])

    def step(c, b, nb):
        wait_gathers(b)

        @pl.when(c + 1 < my_chunks)
        def _():
            wait_idx(nb)
            launch_gathers(nb)

        @pl.when(c + 2 < my_chunks)
        def _():
            issue_idx(c + 2, b)

        compute(c, b)

    pltpu.sync_copy(edges_hbm.at[:, pl.ds(start * C, C)], idx0)
    launch_gathers(0)
    issue_idx(1, 1)

    def pair_body(p, carry):
        step(2 * p, 0, 1)
        step(2 * p + 1, 1, 0)
        return carry

    lax.fori_loop(0, PAIRS, pair_body, 0)

    @pl.when(my_chunks > BASE_CHUNKS)
    def _():
        step(BASE_CHUNKS, 0, 1)

    drain_out(0)
    drain_out(1)


@jax.jit
def _run(table_pk, edge_index):
    mesh = plsc.VectorSubcoreMesh(core_axis_name="c", subcore_axis_name="s")
    kfn = pl.kernel(
        _sc_kernel,
        mesh=mesh,
        compiler_params=pltpu.CompilerParams(
            use_tc_tiling_on_sc=False, needs_layout_passes=False),
        out_type=jax.ShapeDtypeStruct((N_EDGES,), jnp.float32),
        scratch_types=[
            pltpu.VMEM_SHARED((N_NODES, DP), jnp.int32),
            pltpu.VMEM((2, C), jnp.int32),
            pltpu.VMEM((2, C), jnp.int32),
            pltpu.VMEM((C, DP), jnp.int32),
            pltpu.VMEM((C, DP), jnp.int32),
            pltpu.VMEM((C, DP), jnp.int32),
            pltpu.VMEM((C, DP), jnp.int32),
            pltpu.VMEM((C,), jnp.float32),
            pltpu.VMEM((C,), jnp.float32),
            pltpu.SemaphoreType.DMA,
            pltpu.SemaphoreType.DMA,
            pltpu.SemaphoreType.DMA,
            pltpu.SemaphoreType.DMA,
            pltpu.SemaphoreType.DMA,
        ],
    )
    return kfn(table_pk, edge_index)


def kernel(node_embedding_matrix, batch_x_index, edge_index):
    del batch_x_index
    table_bf = node_embedding_matrix.astype(jnp.bfloat16)
    table_pk = jax.lax.bitcast_convert_type(
        table_bf.reshape(N_NODES, DP, 2), jnp.int32)
    return _run(table_pk, edge_index)
